# Initial kernel scaffold; baseline (speedup 1.0000x reference)
import jax
import jax.numpy as jnp
from jax.experimental import pallas as pl


def kernel(x, edge_index, edge_weight, emb, W1, a_src1, a_dst1, b1, gamma, beta, W2, a_src2, a_dst2, b2, Wfc, bfc):
    raise NotImplementedError("write your pallas kernel here")


# speedup vs baseline: 1.7745x; 1.7745x over previous
import jax
import jax.numpy as jnp
from jax import lax
from jax.experimental import pallas as pl
from jax.experimental.pallas import tpu as pltpu

N, E, F_IN = 50000, 800000, 128
HID = 64
NPAD = 50176
EPAD = 802816
BLK = 7168


def _argmax_body(x_ref, o_ref):
    o_ref[...] = jnp.argmax(x_ref[...], axis=1).astype(jnp.int32)


def _l1_body(cnt_ref, idx_ref, e1t_ref, t1_ref, b1_ref, g_ref, be_ref,
             w2_ref, as2_ref, ad2_ref, h2_ref, als_ref, ald_ref):
    idxb = idx_ref[...]
    cols = lax.broadcasted_iota(jnp.int32, (BLK, 128), 1)
    onehot = (idxb[:, None] == cols).astype(jnp.float32)
    ecols = jnp.dot(onehot, e1t_ref[...], preferred_element_type=jnp.float32)
    cnt2 = cnt_ref[...] + onehot
    present = cnt2 > 0
    emax = jnp.max(jnp.where(present, ecols, -1e30), axis=1, keepdims=True)
    ex = jnp.where(present, cnt2 * jnp.exp(ecols - emax), 0.0)
    den = jnp.sum(ex, axis=1, keepdims=True)
    out1 = jnp.dot(ex, t1_ref[...], preferred_element_type=jnp.float32)
    out1 = out1 / (den + 1e-16) + b1_ref[...]
    m = jnp.mean(out1, axis=-1, keepdims=True)
    v = jnp.mean((out1 - m) ** 2, axis=-1, keepdims=True)
    h = (out1 - m) / jnp.sqrt(v + 1e-5) * g_ref[...] + be_ref[...]
    h = jnp.where(h >= 0, h, 0.01 * h)
    h2 = jnp.dot(h, w2_ref[...], preferred_element_type=jnp.float32)
    h2_ref[...] = h2
    als_ref[...] = jnp.sum(h2 * as2_ref[...], axis=1)
    ald_ref[...] = jnp.sum(h2 * ad2_ref[...], axis=1)


def _fin_body(acc_ref, den_ref, h2_ref, als_ref, ald_ref, m2_ref, b2_ref,
              wfct_ref, bfc_ref, o_ref):
    z = als_ref[...] + ald_ref[...]
    wself = jnp.exp(jnp.where(z >= 0, z, 0.2 * z) - m2_ref[0, 0])
    num = acc_ref[...] + wself[:, None] * h2_ref[...]
    d = den_ref[...] + wself
    out2 = num / (d[:, None] + 1e-16) + b2_ref[...]
    h3 = jnp.where(out2 >= 0, out2, 0.01 * out2)
    o_ref[...] = jnp.sum(h3 * wfct_ref[...], axis=1) + bfc_ref[0, 0]


def kernel(x, edge_index, edge_weight, emb, W1, a_src1, a_dst1, b1, gamma,
           beta, W2, a_src2, a_dst2, b2, Wfc, bfc):
    n = x.shape[0]
    f32 = jnp.float32

    T1 = emb @ W1
    A1s = T1 @ a_src1.reshape(-1)
    A1d = T1 @ a_dst1.reshape(-1)
    e1 = A1s[:, None] + A1d[None, :]
    E1 = jnp.where(e1 >= 0, e1, 0.2 * e1)
    E1T = E1.T.astype(f32)

    xp = jnp.pad(x, ((0, NPAD - n), (0, 0)))

    idx = pl.pallas_call(
        _argmax_body,
        grid=(NPAD // BLK,),
        in_specs=[pl.BlockSpec((BLK, 128), lambda i: (i, 0))],
        out_specs=pl.BlockSpec((BLK,), lambda i: (i,)),
        out_shape=jax.ShapeDtypeStruct((NPAD,), jnp.int32),
    )(xp)

    src = edge_index[0].astype(jnp.int32)
    dst = edge_index[1].astype(jnp.int32)
    pad = EPAD - E
    ar = lax.iota(jnp.int32, pad)
    srcp = jnp.concatenate([src, ar % N])
    dstp = jnp.concatenate([dst, N + ar % (NPAD - N)])

    c_e = idx[srcp]
    cnt = jnp.zeros((NPAD * 128,), f32).at[dstp * 128 + c_e].add(1.0)
    cnt = cnt.reshape(NPAD, 128)

    full = lambda a, b: pl.BlockSpec((a, b), lambda i: (0, 0))
    h2, als2, ald2 = pl.pallas_call(
        _l1_body,
        grid=(NPAD // BLK,),
        in_specs=[
            pl.BlockSpec((BLK, 128), lambda i: (i, 0)),
            pl.BlockSpec((BLK,), lambda i: (i,)),
            full(128, 128), full(128, 64), full(1, 64), full(1, 64),
            full(1, 64), full(64, 64), full(1, 64), full(1, 64),
        ],
        out_specs=[
            pl.BlockSpec((BLK, 64), lambda i: (i, 0)),
            pl.BlockSpec((BLK,), lambda i: (i,)),
            pl.BlockSpec((BLK,), lambda i: (i,)),
        ],
        out_shape=[
            jax.ShapeDtypeStruct((NPAD, 64), f32),
            jax.ShapeDtypeStruct((NPAD,), f32),
            jax.ShapeDtypeStruct((NPAD,), f32),
        ],
    )(cnt, idx, E1T, T1.astype(f32), b1.reshape(1, 64), gamma.reshape(1, 64),
      beta.reshape(1, 64), W2.astype(f32), a_src2.reshape(1, 64),
      a_dst2.reshape(1, 64))

    m2 = jnp.max(als2[:n]) + jnp.max(ald2[:n])
    m2 = jnp.where(m2 >= 0, m2, 0.2 * m2).reshape(1, 1)

    z = als2[srcp] + ald2[dstp]
    w = jnp.exp(jnp.where(z >= 0, z, 0.2 * z) - m2[0, 0])
    acc = jnp.zeros((NPAD, 64), f32).at[dstp].add(w[:, None] * h2[srcp])
    den = jnp.zeros((NPAD,), f32).at[dstp].add(w)

    out = pl.pallas_call(
        _fin_body,
        grid=(NPAD // BLK,),
        in_specs=[
            pl.BlockSpec((BLK, 64), lambda i: (i, 0)),
            pl.BlockSpec((BLK,), lambda i: (i,)),
            pl.BlockSpec((BLK, 64), lambda i: (i, 0)),
            pl.BlockSpec((BLK,), lambda i: (i,)),
            pl.BlockSpec((BLK,), lambda i: (i,)),
            pl.BlockSpec(memory_space=pltpu.SMEM),
            full(1, 64), full(1, 64),
            pl.BlockSpec(memory_space=pltpu.SMEM),
        ],
        out_specs=pl.BlockSpec((BLK,), lambda i: (i,)),
        out_shape=jax.ShapeDtypeStruct((NPAD,), f32),
    )(acc, den, h2, als2, ald2, m2, b2.reshape(1, 64),
      Wfc.reshape(1, 64), bfc.reshape(1, 1))

    return out[:n]


# speedup vs baseline: 18.7616x; 10.5727x over previous
import jax
import jax.numpy as jnp
from jax import lax
from jax.experimental import pallas as pl
from jax.experimental.pallas import tpu as pltpu
from jax.experimental.pallas import tpu_sc as plsc

N, E, F_IN = 50000, 800000, 128
HID = 64
NPAD = 50176
EPAD = 802816
BLK = 7168

CH = 12544
CHW = CH * 128
SINK = CHW
EPT = EPAD // 16
NBLK = EPT // 128
STRIPE = CHW // 16


def _hist_body(src_hbm, dst_hbm, idxp_hbm, out_hbm,
               idx_v, srcb, dstb, fidx, fval, zbuf, hist_s):
    cid = lax.axis_index("c")
    sid = lax.axis_index("s")
    lane = lax.iota(jnp.int32, 16)

    pltpu.sync_copy(idxp_hbm, idx_v)

    def zb(i, _):
        zbuf[pl.ds(i * 16, 16)] = jnp.zeros((16,), jnp.float32)
        return 0
    lax.fori_loop(0, 128, zb, 0, unroll=False)

    def chunk_pass(k, _):
        chunk = cid * 2 + k
        base = chunk * CH

        def zcp(i, _):
            pltpu.sync_copy(zbuf, hist_s.at[pl.ds(sid * STRIPE + i * 2048, 2048)])
            return 0
        lax.fori_loop(0, STRIPE // 2048, zcp, 0, unroll=False)
        plsc.subcore_barrier()

        def blk(j, _):
            off = sid * EPT + j * 128
            pltpu.sync_copy(src_hbm.at[pl.ds(off, 128)], srcb)
            pltpu.sync_copy(dst_hbm.at[pl.ds(off, 128)], dstb)
            for q in range(8):
                s16 = srcb[pl.ds(q * 16, 16)]
                d16 = dstb[pl.ds(q * 16, 16)]
                w16 = plsc.load_gather(idx_v, [lax.shift_right_logical(s16, 2)])
                c16 = lax.shift_right_logical(w16, (s16 & 3) * 8) & 0xFF
                loc = d16 - base
                m = (loc >= 0) & (loc < CH)
                fidx[pl.ds(q * 16, 16)] = jnp.where(m, loc * 128 + c16, SINK + lane)
                fval[pl.ds(q * 16, 16)] = jnp.where(m, 1.0, 0.0)
            pltpu.sync_copy(fval, hist_s.at[fidx], add=True)
            return 0
        lax.fori_loop(0, NBLK, blk, 0, unroll=False)
        plsc.subcore_barrier()

        pltpu.sync_copy(hist_s.at[pl.ds(sid * STRIPE, STRIPE)],
                        out_hbm.at[pl.ds(chunk * CHW + sid * STRIPE, STRIPE)])
        plsc.subcore_barrier()
        return 0

    lax.fori_loop(0, 2, chunk_pass, 0, unroll=False)


def _hist_sc(srcp, dstp, idx_packed):
    mesh = plsc.VectorSubcoreMesh(core_axis_name="c", subcore_axis_name="s")
    k = pl.kernel(
        _hist_body,
        out_type=jax.ShapeDtypeStruct((NPAD * 128,), jnp.float32),
        mesh=mesh,
        compiler_params=pltpu.CompilerParams(needs_layout_passes=False),
        scratch_types=[
            pltpu.VMEM((NPAD // 4,), jnp.int32),
            pltpu.VMEM((128,), jnp.int32),
            pltpu.VMEM((128,), jnp.int32),
            pltpu.VMEM((128,), jnp.int32),
            pltpu.VMEM((128,), jnp.float32),
            pltpu.VMEM((2048,), jnp.float32),
            pltpu.VMEM_SHARED((CHW + 16,), jnp.float32),
        ],
    )
    return k(srcp, dstp, idx_packed)


def _argmax_body(x_ref, o_ref):
    o_ref[...] = jnp.argmax(x_ref[...], axis=1).astype(jnp.int32)


def _l1_body(cnt_ref, idx_ref, e1t_ref, t1_ref, b1_ref, g_ref, be_ref,
             w2_ref, as2_ref, ad2_ref, h2_ref, als_ref, ald_ref):
    hi = jax.lax.Precision.HIGHEST
    idxb = idx_ref[...]
    cols = lax.broadcasted_iota(jnp.int32, (BLK, 128), 1)
    onehot = (idxb[:, None] == cols).astype(jnp.float32)
    ecols = jnp.dot(onehot, e1t_ref[...], precision=hi,
                    preferred_element_type=jnp.float32)
    cnt2 = cnt_ref[...] + onehot
    present = cnt2 > 0
    emax = jnp.max(jnp.where(present, ecols, -1e30), axis=1, keepdims=True)
    ex = jnp.where(present, cnt2 * jnp.exp(ecols - emax), 0.0)
    den = jnp.sum(ex, axis=1, keepdims=True)
    out1 = jnp.dot(ex, t1_ref[...], precision=hi,
                   preferred_element_type=jnp.float32)
    out1 = out1 / (den + 1e-16) + b1_ref[...]
    m = jnp.mean(out1, axis=-1, keepdims=True)
    v = jnp.mean((out1 - m) ** 2, axis=-1, keepdims=True)
    h = (out1 - m) / jnp.sqrt(v + 1e-5) * g_ref[...] + be_ref[...]
    h = jnp.where(h >= 0, h, 0.01 * h)
    h2 = jnp.dot(h, w2_ref[...], precision=hi,
                 preferred_element_type=jnp.float32)
    h2_ref[...] = h2
    als_ref[...] = jnp.sum(h2 * as2_ref[...], axis=1)
    ald_ref[...] = jnp.sum(h2 * ad2_ref[...], axis=1)


def _fin_body(acc_ref, den_ref, h2_ref, als_ref, ald_ref, m2_ref, b2_ref,
              wfct_ref, bfc_ref, o_ref):
    z = als_ref[...] + ald_ref[...]
    wself = jnp.exp(jnp.where(z >= 0, z, 0.2 * z) - m2_ref[0, 0])
    num = acc_ref[...] + wself[:, None] * h2_ref[...]
    d = den_ref[...] + wself
    out2 = num / (d[:, None] + 1e-16) + b2_ref[...]
    h3 = jnp.where(out2 >= 0, out2, 0.01 * out2)
    o_ref[...] = jnp.sum(h3 * wfct_ref[...], axis=1) + bfc_ref[0, 0]


NH = NPAD // 2
ASINK = 128
DSINK = 256


def _agg_body(src_hbm, dst_hbm, h2_hbm, als_hbm, ald_hbm, m2_hbm, z_hbm,
              acc_out, den_out,
              srcb, dstb, sb, db, wbuf, didx, didxd, m2b, rowbuf, zrow, stg,
              sem1, sem2, sem3,
              als_s, ald_s, acc_s, den_s):
    cid = lax.axis_index("c")
    sid = lax.axis_index("s")
    lane = lax.iota(jnp.int32, 16)
    base = cid * NH

    pltpu.sync_copy(m2_hbm, m2b)
    pltpu.sync_copy(als_hbm.at[pl.ds(sid * 3136, 3136)], stg)
    pltpu.sync_copy(stg, als_s.at[pl.ds(sid * 3136, 3136)])
    pltpu.sync_copy(ald_hbm.at[pl.ds(sid * 3136, 3136)], stg)
    pltpu.sync_copy(stg, ald_s.at[pl.ds(sid * 3136, 3136)])

    pltpu.sync_copy(z_hbm, zrow)
    def za(i, _):
        pltpu.sync_copy(zrow, acc_s.at[pl.ds(sid * 1568 + i * 32, 32)])
        return 0
    lax.fori_loop(0, 49, za, 0, unroll=False)
    def zd(i, _):
        pltpu.sync_copy(zrow.at[0, pl.ds(0, 32)],
                        den_s.at[pl.ds(sid * 1568 + i * 32, 32)])
        return 0
    lax.fori_loop(0, 49, zd, 0, unroll=False)
    plsc.subcore_barrier()

    m2v = m2b[pl.ds(0, 16)]
    asink = NH + ((sid * 8 + lane) & (ASINK - 1))
    dsink = NH + ((sid * 16 + lane) & (DSINK - 1))

    def blk(j, _):
        off = sid * EPT + j * 128
        pltpu.sync_copy(src_hbm.at[pl.ds(off, 128)], srcb)
        pltpu.sync_copy(dst_hbm.at[pl.ds(off, 128)], dstb)
        g1 = pltpu.async_copy(als_s.at[srcb], sb, sem1)
        g2 = pltpu.async_copy(ald_s.at[dstb], db, sem2)
        g3 = pltpu.async_copy(h2_hbm.at[srcb], rowbuf, sem3)
        g1.wait()
        g2.wait()
        for q in range(8):
            d16 = dstb[pl.ds(q * 16, 16)]
            z = sb[pl.ds(q * 16, 16)] + db[pl.ds(q * 16, 16)]
            e = jnp.where(z >= 0, z, 0.2 * z)
            wbuf[pl.ds(q * 16, 16)] = jnp.exp(e - m2v)
            loc = d16 - base
            m = (loc >= 0) & (loc < NH)
            didx[pl.ds(q * 16, 16)] = jnp.where(m, loc, asink)
            didxd[pl.ds(q * 16, 16)] = jnp.where(m, loc, dsink)
        g3.wait()
        def rs(r, _):
            r16 = jnp.full((16,), r, jnp.int32)
            wv = plsc.load_gather(wbuf, [r16])
            for k in range(4):
                col = lane + (k * 16)
                v = plsc.load_gather(rowbuf, [r16, col])
                plsc.store_scatter(rowbuf, [r16, col], v * wv)
            return 0
        lax.fori_loop(0, 128, rs, 0, unroll=2)
        pltpu.sync_copy(rowbuf, acc_s.at[didx], add=True)
        pltpu.sync_copy(wbuf, den_s.at[didxd], add=True)
        return 0
    lax.fori_loop(0, NBLK, blk, 0, unroll=False)
    plsc.subcore_barrier()

    pltpu.sync_copy(acc_s.at[pl.ds(sid * 1568, 1568)],
                    acc_out.at[pl.ds(cid * NH + sid * 1568, 1568)])
    pltpu.sync_copy(den_s.at[pl.ds(sid * 1568, 1568)],
                    den_out.at[pl.ds(cid * NH + sid * 1568, 1568)])


def _agg_sc(srcp, dstp, h2, als2, ald2, m2x16):
    mesh = plsc.VectorSubcoreMesh(core_axis_name="c", subcore_axis_name="s")
    k = pl.kernel(
        _agg_body,
        out_type=[
            jax.ShapeDtypeStruct((NPAD, 64), jnp.float32),
            jax.ShapeDtypeStruct((NPAD,), jnp.float32),
        ],
        mesh=mesh,
        compiler_params=pltpu.CompilerParams(needs_layout_passes=False,
                                             use_tc_tiling_on_sc=False),
        scratch_types=[
            pltpu.VMEM((128,), jnp.int32),
            pltpu.VMEM((128,), jnp.int32),
            pltpu.VMEM((128,), jnp.float32),
            pltpu.VMEM((128,), jnp.float32),
            pltpu.VMEM((128,), jnp.float32),
            pltpu.VMEM((128,), jnp.int32),
            pltpu.VMEM((128,), jnp.int32),
            pltpu.VMEM((16,), jnp.float32),
            pltpu.VMEM((128, 64), jnp.float32),
            pltpu.VMEM((32, 64), jnp.float32),
            pltpu.VMEM((3136,), jnp.float32),
            pltpu.SemaphoreType.DMA,
            pltpu.SemaphoreType.DMA,
            pltpu.SemaphoreType.DMA,
            pltpu.VMEM_SHARED((NPAD,), jnp.float32),
            pltpu.VMEM_SHARED((NPAD,), jnp.float32),
            pltpu.VMEM_SHARED((NH + ASINK, 64), jnp.float32),
            pltpu.VMEM_SHARED((NH + DSINK,), jnp.float32),
        ],
    )
    zeros = jnp.zeros((32, 64), jnp.float32)
    return k(srcp, dstp, h2, als2, ald2, m2x16, zeros)


def kernel(x, edge_index, edge_weight, emb, W1, a_src1, a_dst1, b1, gamma,
           beta, W2, a_src2, a_dst2, b2, Wfc, bfc):
    n = x.shape[0]
    f32 = jnp.float32

    hi = jax.lax.Precision.HIGHEST
    T1 = jnp.dot(emb, W1, precision=hi)
    A1s = jnp.dot(T1, a_src1.reshape(-1), precision=hi)
    A1d = jnp.dot(T1, a_dst1.reshape(-1), precision=hi)
    e1 = A1s[:, None] + A1d[None, :]
    E1 = jnp.where(e1 >= 0, e1, 0.2 * e1)
    E1T = E1.T.astype(f32)

    xp = jnp.pad(x, ((0, NPAD - n), (0, 0)))

    idx = pl.pallas_call(
        _argmax_body,
        grid=(NPAD // BLK,),
        in_specs=[pl.BlockSpec((BLK, 128), lambda i: (i, 0))],
        out_specs=pl.BlockSpec((BLK,), lambda i: (i,)),
        out_shape=jax.ShapeDtypeStruct((NPAD,), jnp.int32),
    )(xp)

    src = edge_index[0].astype(jnp.int32)
    dst = edge_index[1].astype(jnp.int32)
    pad = EPAD - E
    ar = lax.iota(jnp.int32, pad)
    srcp = jnp.concatenate([src, ar % N])
    dstp = jnp.concatenate([dst, N + ar % (NPAD - N)])

    i4 = idx.reshape(NPAD // 4, 4)
    idx_packed = (i4[:, 0] | (i4[:, 1] << 8) | (i4[:, 2] << 16)
                  | (i4[:, 3] << 24))
    cnt = _hist_sc(srcp, dstp, idx_packed).reshape(NPAD, 128)

    full = lambda a, b: pl.BlockSpec((a, b), lambda i: (0, 0))
    h2, als2, ald2 = pl.pallas_call(
        _l1_body,
        grid=(NPAD // BLK,),
        in_specs=[
            pl.BlockSpec((BLK, 128), lambda i: (i, 0)),
            pl.BlockSpec((BLK,), lambda i: (i,)),
            full(128, 128), full(128, 64), full(1, 64), full(1, 64),
            full(1, 64), full(64, 64), full(1, 64), full(1, 64),
        ],
        out_specs=[
            pl.BlockSpec((BLK, 64), lambda i: (i, 0)),
            pl.BlockSpec((BLK,), lambda i: (i,)),
            pl.BlockSpec((BLK,), lambda i: (i,)),
        ],
        out_shape=[
            jax.ShapeDtypeStruct((NPAD, 64), f32),
            jax.ShapeDtypeStruct((NPAD,), f32),
            jax.ShapeDtypeStruct((NPAD,), f32),
        ],
    )(cnt, idx, E1T, T1.astype(f32), b1.reshape(1, 64), gamma.reshape(1, 64),
      beta.reshape(1, 64), W2.astype(f32), a_src2.reshape(1, 64),
      a_dst2.reshape(1, 64))

    m2 = jnp.max(als2[:n]) + jnp.max(ald2[:n])
    m2 = jnp.where(m2 >= 0, m2, 0.2 * m2).reshape(1, 1)

    m2x16 = jnp.full((16,), m2[0, 0], f32)
    acc, den = _agg_sc(srcp, dstp, h2, als2, ald2, m2x16)

    out = pl.pallas_call(
        _fin_body,
        grid=(NPAD // BLK,),
        in_specs=[
            pl.BlockSpec((BLK, 64), lambda i: (i, 0)),
            pl.BlockSpec((BLK,), lambda i: (i,)),
            pl.BlockSpec((BLK, 64), lambda i: (i, 0)),
            pl.BlockSpec((BLK,), lambda i: (i,)),
            pl.BlockSpec((BLK,), lambda i: (i,)),
            pl.BlockSpec(memory_space=pltpu.SMEM),
            full(1, 64), full(1, 64),
            pl.BlockSpec(memory_space=pltpu.SMEM),
        ],
        out_specs=pl.BlockSpec((BLK,), lambda i: (i,)),
        out_shape=jax.ShapeDtypeStruct((NPAD,), f32),
    )(acc, den, h2, als2, ald2, m2, b2.reshape(1, 64),
      Wfc.reshape(1, 64), bfc.reshape(1, 1))

    return out[:n]
